# baseline (speedup 1.0000x reference)
import functools

import jax
import jax.numpy as jnp
from jax import lax
from jax.experimental import pallas as pl
from jax.experimental.pallas import tpu as pltpu
from jax.experimental.pallas import tpu_sc as plsc

N = 10000
E = 320000
D = 128
EDIM = 16
C = 128
HEADS = [8, 8, 1]
NEG_SLOPE = 0.2
EPS = 1e-5

NC = 2
NS = 16
NW = NC * NS
ET = 10368
EPAD = NW * ET
NEGBIG = -1e9
HP = 16
CHA = 128
CHB = 96
ROWS = 1000


@functools.lru_cache(maxsize=None)
def _mesh():
    return plsc.VectorSubcoreMesh(
        core_axis_name="c", subcore_axis_name="s",
        num_cores=NC, num_subcores=NS)


_RB = 1000


def _split_heads(xw, nheads):
    lo = jnp.concatenate([xw[:, h * C:h * C + 64] for h in range(nheads)], axis=1)
    hi = jnp.concatenate([xw[:, h * C + 64:(h + 1) * C] for h in range(nheads)], axis=1)
    return lo, hi


def _mm_body(nheads, h_ref, w_ref, am_ref, dm_ref, lo_ref, hi_ref, a_ref, d_ref):
    xw = jnp.dot(h_ref[...], w_ref[...], preferred_element_type=jnp.float32)
    lo_ref[...], hi_ref[...] = _split_heads(xw, nheads)
    a_ref[...] = jnp.dot(xw, am_ref[...], preferred_element_type=jnp.float32)
    d_ref[...] = jnp.dot(xw, dm_ref[...], preferred_element_type=jnp.float32)


def _tc_mm(h, w, am, dm):
    hc = w.shape[1]
    return pl.pallas_call(
        functools.partial(_mm_body, hc // C),
        grid=(N // _RB,),
        in_specs=[
            pl.BlockSpec((_RB, D), lambda i: (i, 0)),
            pl.BlockSpec((D, hc), lambda i: (0, 0)),
            pl.BlockSpec((hc, HP), lambda i: (0, 0)),
            pl.BlockSpec((hc, HP), lambda i: (0, 0)),
        ],
        out_specs=[
            pl.BlockSpec((_RB, hc // 2), lambda i: (i, 0)),
            pl.BlockSpec((_RB, hc // 2), lambda i: (i, 0)),
            pl.BlockSpec((_RB, HP), lambda i: (i, 0)),
            pl.BlockSpec((_RB, HP), lambda i: (i, 0)),
        ],
        out_shape=[
            jax.ShapeDtypeStruct((N, hc // 2), jnp.float32),
            jax.ShapeDtypeStruct((N, hc // 2), jnp.float32),
            jax.ShapeDtypeStruct((N, HP), jnp.float32),
            jax.ShapeDtypeStruct((N, HP), jnp.float32),
        ],
    )(h, w, am, dm)


def _init_body(nheads, x_ref, iw_ref, ib_ref, w_ref, am_ref, dm_ref,
               h_ref, lo_ref, hi_ref, a_ref, d_ref):
    h = jnp.dot(x_ref[...], iw_ref[...], preferred_element_type=jnp.float32)
    h = h + ib_ref[...]
    h_ref[...] = h
    xw = jnp.dot(h, w_ref[...], preferred_element_type=jnp.float32)
    lo_ref[...], hi_ref[...] = _split_heads(xw, nheads)
    a_ref[...] = jnp.dot(xw, am_ref[...], preferred_element_type=jnp.float32)
    d_ref[...] = jnp.dot(xw, dm_ref[...], preferred_element_type=jnp.float32)


def _tc_init(x, iw, ib, w, am, dm):
    hc = w.shape[1]
    return pl.pallas_call(
        functools.partial(_init_body, hc // C),
        grid=(N // _RB,),
        in_specs=[
            pl.BlockSpec((_RB, D), lambda i: (i, 0)),
            pl.BlockSpec((D, C), lambda i: (0, 0)),
            pl.BlockSpec((1, C), lambda i: (0, 0)),
            pl.BlockSpec((C, hc), lambda i: (0, 0)),
            pl.BlockSpec((hc, HP), lambda i: (0, 0)),
            pl.BlockSpec((hc, HP), lambda i: (0, 0)),
        ],
        out_specs=[
            pl.BlockSpec((_RB, C), lambda i: (i, 0)),
            pl.BlockSpec((_RB, hc // 2), lambda i: (i, 0)),
            pl.BlockSpec((_RB, hc // 2), lambda i: (i, 0)),
            pl.BlockSpec((_RB, HP), lambda i: (i, 0)),
            pl.BlockSpec((_RB, HP), lambda i: (i, 0)),
        ],
        out_shape=[
            jax.ShapeDtypeStruct((N, C), jnp.float32),
            jax.ShapeDtypeStruct((N, hc // 2), jnp.float32),
            jax.ShapeDtypeStruct((N, hc // 2), jnp.float32),
            jax.ShapeDtypeStruct((N, HP), jnp.float32),
            jax.ShapeDtypeStruct((N, HP), jnp.float32),
        ],
    )(x, iw, ib.reshape(1, C), w, am, dm)


_AEB = 2000


def _ae_body(ea_ref, wr_ref, mask_ref, ae_ref, sum_ref):
    o = jnp.dot(ea_ref[...], wr_ref[...], preferred_element_type=jnp.float32)
    o = o + mask_ref[...]
    ae_ref[...] = o

    @pl.when(pl.program_id(0) == 0)
    def _():
        sum_ref[...] = jnp.zeros_like(sum_ref)

    sum_ref[...] += jnp.sum(o, axis=0, keepdims=True)


def _tc_ae(ea, wr_cat, mask_cat):
    w3 = wr_cat.shape[1]
    return pl.pallas_call(
        _ae_body,
        grid=(E // _AEB,),
        in_specs=[
            pl.BlockSpec((_AEB, EDIM), lambda i: (i, 0)),
            pl.BlockSpec((EDIM, w3), lambda i: (0, 0)),
            pl.BlockSpec((1, w3), lambda i: (0, 0)),
        ],
        out_specs=[
            pl.BlockSpec((_AEB, w3), lambda i: (i, 0)),
            pl.BlockSpec((1, w3), lambda i: (0, 0)),
        ],
        out_shape=[
            jax.ShapeDtypeStruct((E, w3), jnp.float32),
            jax.ShapeDtypeStruct((1, w3), jnp.float32),
        ],
    )(ea, wr_cat, mask_cat.reshape(1, w3))


_DB = 16000


def _densum_body(dp_ref, inv_ref):
    s = jnp.sum(dp_ref[...], axis=0)
    inv_ref[...] = 1.0 / (s + 1e-16)


def _tc_densum(dparts):
    return pl.pallas_call(
        _densum_body,
        grid=(1,),
        in_specs=[pl.BlockSpec((NW, N * 8), lambda i: (0, 0))],
        out_specs=pl.BlockSpec((N * 8,), lambda i: (0,)),
        out_shape=jax.ShapeDtypeStruct((N * 8,), jnp.float32),
    )(dparts)


def _combine_body(inv_h, p0_ref, p1_ref, res_ref, b_ref, g_ref, bb_ref,
                  out_ref):
    o = (p0_ref[...] + p1_ref[...]) * inv_h + b_ref[...]
    mu = jnp.mean(o, axis=0, keepdims=True)
    var = jnp.mean(o * o, axis=0, keepdims=True) - mu * mu
    o = (o - mu) * lax.rsqrt(var + EPS) * g_ref[...] + bb_ref[...]
    o = jnp.where(o > 0, o, jnp.exp(o) - 1.0)
    out_ref[...] = o + res_ref[...]


def _tc_combine(p0, p1, res, b, g, bb, nheads):
    body = functools.partial(_combine_body, 1.0 / nheads)
    return pl.pallas_call(
        body,
        grid=(1,),
        in_specs=[
            pl.BlockSpec((N, C), lambda i: (0, 0)),
            pl.BlockSpec((N, C), lambda i: (0, 0)),
            pl.BlockSpec((N, C), lambda i: (0, 0)),
            pl.BlockSpec((1, C), lambda i: (0, 0)),
            pl.BlockSpec((1, C), lambda i: (0, 0)),
            pl.BlockSpec((1, C), lambda i: (0, 0)),
        ],
        out_specs=pl.BlockSpec((N, C), lambda i: (0, 0)),
        out_shape=jax.ShapeDtypeStruct((N, C), jnp.float32),
    )(p0, p1, res, b.reshape(1, C), g.reshape(1, C), bb.reshape(1, C))


def _passA_body(src_h, dst_h, d8_h, as_h, ad_h, ae_h, z8_h,
                ex_h, dp_h,
                sidx, didx, d8buf, buf_s, buf_d, buf_e, den8, sem):
    c = lax.axis_index("c")
    s = lax.axis_index("s")
    wid = c * NS + s
    pltpu.sync_copy(z8_h, den8)
    lanes = lax.iota(jnp.int32, 16)
    msk8 = lanes < 8

    def chunk(ch, carry):
        base = wid * ET + ch * CHA
        pltpu.sync_copy(src_h.at[pl.ds(base, CHA)], sidx)
        pltpu.sync_copy(dst_h.at[pl.ds(base, CHA)], didx)
        pltpu.sync_copy(d8_h.at[pl.ds(base * 8, CHA * 8)],
                        d8buf.at[pl.ds(0, CHA * 8)])
        cp1 = pltpu.async_copy(as_h.at[sidx], buf_s, sem)
        cp2 = pltpu.async_copy(ad_h.at[didx], buf_d, sem)
        pltpu.sync_copy(ae_h.at[pl.ds(base, CHA)], buf_e)
        cp1.wait()
        cp2.wait()

        def edge(e, carry2):
            a = buf_s[e, :] + buf_d[e, :] + buf_e[e, :]
            a = jnp.where(a >= 0, a, jnp.float32(NEG_SLOPE) * a)
            ex = jnp.exp(a)
            buf_e[e, :] = ex
            idxd = d8buf[pl.ds(e * 8, 16)]
            plsc.addupdate_scatter(den8, [idxd], ex, mask=msk8)
            return carry2

        lax.fori_loop(0, CHA, edge, 0)
        pltpu.sync_copy(buf_e, ex_h.at[pl.ds(base, CHA)])
        return carry

    lax.fori_loop(0, ET // CHA, chunk, 0)
    pltpu.sync_copy(den8, dp_h.at[wid])


@functools.lru_cache(maxsize=None)
def _make_passA():
    return pl.kernel(
        _passA_body,
        mesh=_mesh(),
        out_type=[
            jax.ShapeDtypeStruct((EPAD, HP), jnp.float32),
            jax.ShapeDtypeStruct((NW, N * 8), jnp.float32),
        ],
        scratch_types=[
            pltpu.VMEM((CHA,), jnp.int32),
            pltpu.VMEM((CHA,), jnp.int32),
            pltpu.VMEM((CHA * 8 + 16,), jnp.int32),
            pltpu.VMEM((CHA, HP), jnp.float32),
            pltpu.VMEM((CHA, HP), jnp.float32),
            pltpu.VMEM((CHA, HP), jnp.float32),
            pltpu.VMEM((N * 8,), jnp.float32),
            pltpu.SemaphoreType.DMA,
        ],
        compiler_params=pltpu.CompilerParams(use_tc_tiling_on_sc=False, needs_layout_passes=False),
    )


@functools.lru_cache(maxsize=None)
def _make_passB(nheads, chb, cw, nph):
    hw = nheads * cw
    nch = ET // chb

    def body(src_h, dst_h, xw_h, ex_h, inv_h, z_h, p_h,
             sidx0, didx0, xwbuf0, invb0, exb0,
             sidx1, didx1, xwbuf1, invb1, exb1,
             outb, out_sh, sem0, sem1):
        c = lax.axis_index("c")
        s = lax.axis_index("s")
        wid = c * NS + s
        bufs = ((sidx0, didx0, xwbuf0, invb0, exb0),
                (sidx1, didx1, xwbuf1, invb1, exb1))
        sems = (sem0, sem1)

        def stage(k, ch, par):
            sidx, didx, xwbuf, invb, exb = bufs[par]
            base = wid * ET + ch * chb
            pltpu.sync_copy(src_h.at[pl.ds(base, chb)], sidx)
            pltpu.sync_copy(dst_h.at[pl.ds(base, chb)], didx)
            kn = k * N

            def addk(j, carry):
                sl = pl.ds(j * 16, 16)
                sidx[sl] = sidx[sl] + kn
                return carry

            lax.fori_loop(0, chb // 16, addk, 0)
            pltpu.async_copy(xw_h.at[sidx], xwbuf, sems[par])
            pltpu.async_copy(inv_h.at[didx], invb, sems[par])
            pltpu.async_copy(ex_h.at[pl.ds(base, chb)], exb, sems[par])

        def drain(par):
            sidx, didx, xwbuf, invb, exb = bufs[par]
            pltpu.make_async_copy(xw_h.at[sidx], xwbuf, sems[par]).wait()
            pltpu.make_async_copy(inv_h.at[didx], invb, sems[par]).wait()
            pltpu.make_async_copy(ex_h.at[pl.ds(0, chb)], exb, sems[par]).wait()

        def compute(par):
            sidx, didx, xwbuf, invb, exb = bufs[par]

            def edge(e, carry2):
                w = exb[e, :] * invb[e, :]
                for ci in range(cw // 16):
                    acc = w[0] * xwbuf[e, pl.ds(ci * 16, 16)]
                    for hh in range(1, nheads):
                        acc = acc + w[hh] * xwbuf[e, pl.ds(hh * cw + ci * 16, 16)]
                    outb[e, pl.ds(ci * 16, 16)] = acc
                return carry2

            lax.fori_loop(0, chb, edge, 0)
            pltpu.sync_copy(outb, out_sh.at[didx], add=True)

        def phase(k, carry):
            @pl.when(s < N // ROWS)
            def _():
                sl = pl.ds(s * ROWS, ROWS)
                pltpu.sync_copy(z_h.at[sl], out_sh.at[sl])

            plsc.subcore_barrier()
            stage(k, 0, 0)

            def pair(ch2, carry2):
                for par in (0, 1):
                    ch = 2 * ch2 + par

                    @pl.when(ch + 1 < nch)
                    def _():
                        stage(k, ch + 1, 1 - par)

                    drain(par)
                    compute(par)
                return carry2

            lax.fori_loop(0, nch // 2, pair, 0)
            plsc.subcore_barrier()

            @pl.when(s < N // ROWS)
            def _():
                off = (2 * k + c) * N + s * ROWS
                pltpu.sync_copy(out_sh.at[pl.ds(s * ROWS, ROWS)],
                                p_h.at[pl.ds(off, ROWS)])

            plsc.subcore_barrier()
            return carry

        lax.fori_loop(0, nph, phase, 0)

    dbuf = [
        pltpu.VMEM((chb,), jnp.int32),
        pltpu.VMEM((chb,), jnp.int32),
        pltpu.VMEM((chb, hw), jnp.float32),
        pltpu.VMEM((chb, HP), jnp.float32),
        pltpu.VMEM((chb, HP), jnp.float32),
    ]
    return pl.kernel(
        body,
        mesh=_mesh(),
        out_type=jax.ShapeDtypeStruct((nph * 2 * N, cw), jnp.float32),
        scratch_types=dbuf + dbuf + [
            pltpu.VMEM((chb, cw), jnp.float32),
            pltpu.VMEM_SHARED((N, cw), jnp.float32),
            pltpu.SemaphoreType.DMA,
            pltpu.SemaphoreType.DMA,
        ],
        compiler_params=pltpu.CompilerParams(use_tc_tiling_on_sc=False, needs_layout_passes=False),
    )


def _head_matrix(a, nheads):
    m = jnp.zeros((nheads * C, HP), jnp.float32)
    idx = jnp.arange(nheads * C)
    return m.at[idx, idx // C].set(a.reshape(-1))


def kernel(x, edge_index, edge_attr, params):
    ei = edge_index.astype(jnp.int32)
    loop = jnp.arange(N, dtype=jnp.int32)
    padn = EPAD - E - N
    src = jnp.concatenate([ei[0], loop, jnp.zeros((padn,), jnp.int32)])
    dst = jnp.concatenate([ei[1], loop, jnp.zeros((padn,), jnp.int32)])
    dst8 = (dst[:, None] * 8 + jnp.arange(8, dtype=jnp.int32)).reshape(-1)

    wrs, masks = [], []
    for i, h in enumerate(HEADS):
        we = params['We%d' % i]
        aep = params['ae%d' % i]
        wr = (we.reshape(EDIM, h, C) * aep[None]).sum(-1)
        wrs.append(jnp.pad(wr, ((0, 0), (0, HP - h))))
        masks.append(jnp.where(jnp.arange(HP) < h, 0.0, NEGBIG).astype(jnp.float32))
    wr_cat = jnp.concatenate(wrs, axis=1)
    mask_cat = jnp.concatenate(masks)

    ae_real, ae_sum = _tc_ae(edge_attr, wr_cat, mask_cat)
    ae_self = ae_sum[0] / E
    ae_full = jnp.concatenate([
        ae_real,
        jnp.broadcast_to(ae_self, (N, 3 * HP)),
        jnp.full((padn, 3 * HP), NEGBIG, jnp.float32),
    ], axis=0)

    z8 = jnp.zeros((N * 8,), jnp.float32)
    z32 = jnp.zeros((N, 32), jnp.float32)

    h = None
    xw = a_t = d_t = None
    for i, nh in enumerate(HEADS):
        am = _head_matrix(params['as%d' % i], nh)
        dm = _head_matrix(params['ad%d' % i], nh)
        if i == 0:
            h, xwlo, xwhi, a_t, d_t = _tc_init(x, params['in_W'], params['in_b'],
                                               params['W0'], am, dm)
        else:
            xwlo, xwhi, a_t, d_t = _tc_mm(h, params['W%d' % i], am, dm)

        ae_l = lax.slice_in_dim(ae_full, i * HP, (i + 1) * HP, axis=1)
        ex, dparts = _make_passA()(src, dst, dst8, a_t, d_t, ae_l, z8)
        inv8 = _tc_densum(dparts).reshape(N, 8)
        inv = jnp.concatenate([inv8, jnp.zeros((N, 8), jnp.float32)], axis=1)
        nph, cw = 4, 32
        if nh == 8:
            tabs = [jnp.concatenate(
                        [xwlo[:, 64 * hh + 32 * j:64 * hh + 32 * j + 32]
                         for hh in range(8)], axis=1) for j in range(2)]
            tabs += [jnp.concatenate(
                        [xwhi[:, 64 * hh + 32 * j:64 * hh + 32 * j + 32]
                         for hh in range(8)], axis=1) for j in range(2)]
        else:
            tabs = [t[:, 32 * j:32 * j + 32] for t in (xwlo, xwhi)
                    for j in range(2)]
        xw_cat = jnp.concatenate(tabs, axis=0)
        zz = z32
        p_cat = _make_passB(nh, CHB, cw, nph)(src, dst, xw_cat, ex, inv, zz)
        p0 = jnp.concatenate(
            [p_cat[2 * k * N:(2 * k + 1) * N] for k in range(nph)], axis=1)
        p1 = jnp.concatenate(
            [p_cat[(2 * k + 1) * N:(2 * k + 2) * N] for k in range(nph)], axis=1)
        h = _tc_combine(p0, p1, h, params['b%d' % i],
                        params['g%d' % i], params['beta%d' % i], nh)
    return h


# speedup vs baseline: 1.0447x; 1.0447x over previous
import functools

import jax
import jax.numpy as jnp
from jax import lax
from jax.experimental import pallas as pl
from jax.experimental.pallas import tpu as pltpu
from jax.experimental.pallas import tpu_sc as plsc

N = 10000
E = 320000
D = 128
EDIM = 16
C = 128
HEADS = [8, 8, 1]
NEG_SLOPE = 0.2
EPS = 1e-5

NC = 2
NS = 16
NW = NC * NS
ET = 10368
EPAD = NW * ET
NEGBIG = -1e9
HP = 16
CHA = 96
CHB = 96
ROWS = 1000


@functools.lru_cache(maxsize=None)
def _mesh():
    return plsc.VectorSubcoreMesh(
        core_axis_name="c", subcore_axis_name="s",
        num_cores=NC, num_subcores=NS)


_RB = 1000


def _split_heads(xw, nheads):
    lo = jnp.concatenate([xw[:, h * C:h * C + 64] for h in range(nheads)], axis=1)
    hi = jnp.concatenate([xw[:, h * C + 64:(h + 1) * C] for h in range(nheads)], axis=1)
    return lo, hi


def _mm_body(nheads, h_ref, w_ref, am_ref, dm_ref, lo_ref, hi_ref, a_ref, d_ref):
    xw = jnp.dot(h_ref[...], w_ref[...], preferred_element_type=jnp.float32)
    lo_ref[...], hi_ref[...] = _split_heads(xw, nheads)
    a_ref[...] = jnp.dot(xw, am_ref[...], preferred_element_type=jnp.float32)
    d_ref[...] = jnp.dot(xw, dm_ref[...], preferred_element_type=jnp.float32)


def _tc_mm(h, w, am, dm):
    hc = w.shape[1]
    return pl.pallas_call(
        functools.partial(_mm_body, hc // C),
        grid=(N // _RB,),
        in_specs=[
            pl.BlockSpec((_RB, D), lambda i: (i, 0)),
            pl.BlockSpec((D, hc), lambda i: (0, 0)),
            pl.BlockSpec((hc, HP), lambda i: (0, 0)),
            pl.BlockSpec((hc, HP), lambda i: (0, 0)),
        ],
        out_specs=[
            pl.BlockSpec((_RB, hc // 2), lambda i: (i, 0)),
            pl.BlockSpec((_RB, hc // 2), lambda i: (i, 0)),
            pl.BlockSpec((_RB, HP), lambda i: (i, 0)),
            pl.BlockSpec((_RB, HP), lambda i: (i, 0)),
        ],
        out_shape=[
            jax.ShapeDtypeStruct((N, hc // 2), jnp.float32),
            jax.ShapeDtypeStruct((N, hc // 2), jnp.float32),
            jax.ShapeDtypeStruct((N, HP), jnp.float32),
            jax.ShapeDtypeStruct((N, HP), jnp.float32),
        ],
    )(h, w, am, dm)


def _init_body(nheads, x_ref, iw_ref, ib_ref, w_ref, am_ref, dm_ref,
               h_ref, lo_ref, hi_ref, a_ref, d_ref):
    h = jnp.dot(x_ref[...], iw_ref[...], preferred_element_type=jnp.float32)
    h = h + ib_ref[...]
    h_ref[...] = h
    xw = jnp.dot(h, w_ref[...], preferred_element_type=jnp.float32)
    lo_ref[...], hi_ref[...] = _split_heads(xw, nheads)
    a_ref[...] = jnp.dot(xw, am_ref[...], preferred_element_type=jnp.float32)
    d_ref[...] = jnp.dot(xw, dm_ref[...], preferred_element_type=jnp.float32)


def _tc_init(x, iw, ib, w, am, dm):
    hc = w.shape[1]
    return pl.pallas_call(
        functools.partial(_init_body, hc // C),
        grid=(N // _RB,),
        in_specs=[
            pl.BlockSpec((_RB, D), lambda i: (i, 0)),
            pl.BlockSpec((D, C), lambda i: (0, 0)),
            pl.BlockSpec((1, C), lambda i: (0, 0)),
            pl.BlockSpec((C, hc), lambda i: (0, 0)),
            pl.BlockSpec((hc, HP), lambda i: (0, 0)),
            pl.BlockSpec((hc, HP), lambda i: (0, 0)),
        ],
        out_specs=[
            pl.BlockSpec((_RB, C), lambda i: (i, 0)),
            pl.BlockSpec((_RB, hc // 2), lambda i: (i, 0)),
            pl.BlockSpec((_RB, hc // 2), lambda i: (i, 0)),
            pl.BlockSpec((_RB, HP), lambda i: (i, 0)),
            pl.BlockSpec((_RB, HP), lambda i: (i, 0)),
        ],
        out_shape=[
            jax.ShapeDtypeStruct((N, C), jnp.float32),
            jax.ShapeDtypeStruct((N, hc // 2), jnp.float32),
            jax.ShapeDtypeStruct((N, hc // 2), jnp.float32),
            jax.ShapeDtypeStruct((N, HP), jnp.float32),
            jax.ShapeDtypeStruct((N, HP), jnp.float32),
        ],
    )(x, iw, ib.reshape(1, C), w, am, dm)


_AEB = 2000


def _ae_body(ea_ref, wr_ref, mask_ref, ae_ref, sum_ref):
    o = jnp.dot(ea_ref[...], wr_ref[...], preferred_element_type=jnp.float32)
    o = o + mask_ref[...]
    ae_ref[...] = o

    @pl.when(pl.program_id(0) == 0)
    def _():
        sum_ref[...] = jnp.zeros_like(sum_ref)

    sum_ref[...] += jnp.sum(o, axis=0, keepdims=True)


def _tc_ae(ea, wr_cat, mask_cat):
    w3 = wr_cat.shape[1]
    return pl.pallas_call(
        _ae_body,
        grid=(E // _AEB,),
        in_specs=[
            pl.BlockSpec((_AEB, EDIM), lambda i: (i, 0)),
            pl.BlockSpec((EDIM, w3), lambda i: (0, 0)),
            pl.BlockSpec((1, w3), lambda i: (0, 0)),
        ],
        out_specs=[
            pl.BlockSpec((_AEB, w3), lambda i: (i, 0)),
            pl.BlockSpec((1, w3), lambda i: (0, 0)),
        ],
        out_shape=[
            jax.ShapeDtypeStruct((E, w3), jnp.float32),
            jax.ShapeDtypeStruct((1, w3), jnp.float32),
        ],
    )(ea, wr_cat, mask_cat.reshape(1, w3))


_DB = 16000


def _densum_body(dp_ref, inv_ref):
    s = jnp.sum(dp_ref[...], axis=0)
    inv_ref[...] = 1.0 / (s + 1e-16)


def _tc_densum(dparts):
    return pl.pallas_call(
        _densum_body,
        grid=(1,),
        in_specs=[pl.BlockSpec((NW, N * 8), lambda i: (0, 0))],
        out_specs=pl.BlockSpec((N * 8,), lambda i: (0,)),
        out_shape=jax.ShapeDtypeStruct((N * 8,), jnp.float32),
    )(dparts)


def _combine_body(inv_h, p0_ref, p1_ref, res_ref, b_ref, g_ref, bb_ref,
                  out_ref):
    o = (p0_ref[...] + p1_ref[...]) * inv_h + b_ref[...]
    mu = jnp.mean(o, axis=0, keepdims=True)
    var = jnp.mean(o * o, axis=0, keepdims=True) - mu * mu
    o = (o - mu) * lax.rsqrt(var + EPS) * g_ref[...] + bb_ref[...]
    o = jnp.where(o > 0, o, jnp.exp(o) - 1.0)
    out_ref[...] = o + res_ref[...]


def _tc_combine(p0, p1, res, b, g, bb, nheads):
    body = functools.partial(_combine_body, 1.0 / nheads)
    return pl.pallas_call(
        body,
        grid=(1,),
        in_specs=[
            pl.BlockSpec((N, C), lambda i: (0, 0)),
            pl.BlockSpec((N, C), lambda i: (0, 0)),
            pl.BlockSpec((N, C), lambda i: (0, 0)),
            pl.BlockSpec((1, C), lambda i: (0, 0)),
            pl.BlockSpec((1, C), lambda i: (0, 0)),
            pl.BlockSpec((1, C), lambda i: (0, 0)),
        ],
        out_specs=pl.BlockSpec((N, C), lambda i: (0, 0)),
        out_shape=jax.ShapeDtypeStruct((N, C), jnp.float32),
    )(p0, p1, res, b.reshape(1, C), g.reshape(1, C), bb.reshape(1, C))


def _passA_body(src_h, dst_h, d8_h, as_h, ad_h, ae_h, z8_h,
                ex_h, dp_h,
                sidx0, didx0, d8b0, bs0, bd0, be0,
                sidx1, didx1, d8b1, bs1, bd1, be1,
                den8, sem0, sem1):
    c = lax.axis_index("c")
    s = lax.axis_index("s")
    wid = c * NS + s
    bufs = ((sidx0, didx0, d8b0, bs0, bd0, be0),
            (sidx1, didx1, d8b1, bs1, bd1, be1))
    sems = (sem0, sem1)
    nch = ET // CHA
    pltpu.sync_copy(z8_h, den8)
    lanes = lax.iota(jnp.int32, 16)
    msk8 = lanes < 8

    def stage(ch, par):
        sidx, didx, d8buf, buf_s, buf_d, buf_e = bufs[par]
        base = wid * ET + ch * CHA
        pltpu.sync_copy(src_h.at[pl.ds(base, CHA)], sidx)
        pltpu.sync_copy(dst_h.at[pl.ds(base, CHA)], didx)
        pltpu.async_copy(d8_h.at[pl.ds(base * 8, CHA * 8)],
                         d8buf.at[pl.ds(0, CHA * 8)], sems[par])
        pltpu.async_copy(as_h.at[sidx], buf_s, sems[par])
        pltpu.async_copy(ad_h.at[didx], buf_d, sems[par])
        pltpu.async_copy(ae_h.at[pl.ds(base, CHA)], buf_e, sems[par])

    def drain(par):
        sidx, didx, d8buf, buf_s, buf_d, buf_e = bufs[par]
        pltpu.make_async_copy(d8_h.at[pl.ds(0, CHA * 8)],
                              d8buf.at[pl.ds(0, CHA * 8)], sems[par]).wait()
        pltpu.make_async_copy(as_h.at[sidx], buf_s, sems[par]).wait()
        pltpu.make_async_copy(ad_h.at[didx], buf_d, sems[par]).wait()
        pltpu.make_async_copy(ae_h.at[pl.ds(0, CHA)], buf_e, sems[par]).wait()

    def compute(ch, par):
        sidx, didx, d8buf, buf_s, buf_d, buf_e = bufs[par]
        base = wid * ET + ch * CHA

        def edge(e, carry2):
            a = buf_s[e, :] + buf_d[e, :] + buf_e[e, :]
            a = jnp.where(a >= 0, a, jnp.float32(NEG_SLOPE) * a)
            ex = jnp.exp(a)
            buf_e[e, :] = ex
            idxd = d8buf[pl.ds(e * 8, 16)]
            plsc.addupdate_scatter(den8, [idxd], ex, mask=msk8)
            return carry2

        lax.fori_loop(0, CHA, edge, 0)
        pltpu.sync_copy(buf_e, ex_h.at[pl.ds(base, CHA)])

    stage(0, 0)

    def pair(ch2, carry):
        for par in (0, 1):
            ch = 2 * ch2 + par

            @pl.when(ch + 1 < nch)
            def _():
                stage(ch + 1, 1 - par)

            drain(par)
            compute(ch, par)
        return carry

    lax.fori_loop(0, nch // 2, pair, 0)
    pltpu.sync_copy(den8, dp_h.at[wid])


@functools.lru_cache(maxsize=None)
def _make_passA():
    abuf = [
        pltpu.VMEM((CHA,), jnp.int32),
        pltpu.VMEM((CHA,), jnp.int32),
        pltpu.VMEM((CHA * 8 + 16,), jnp.int32),
        pltpu.VMEM((CHA, HP), jnp.float32),
        pltpu.VMEM((CHA, HP), jnp.float32),
        pltpu.VMEM((CHA, HP), jnp.float32),
    ]
    return pl.kernel(
        _passA_body,
        mesh=_mesh(),
        out_type=[
            jax.ShapeDtypeStruct((EPAD, HP), jnp.float32),
            jax.ShapeDtypeStruct((NW, N * 8), jnp.float32),
        ],
        scratch_types=abuf + abuf + [
            pltpu.VMEM((N * 8,), jnp.float32),
            pltpu.SemaphoreType.DMA,
            pltpu.SemaphoreType.DMA,
        ],
        compiler_params=pltpu.CompilerParams(use_tc_tiling_on_sc=False, needs_layout_passes=False),
    )


@functools.lru_cache(maxsize=None)
def _make_passB(nheads, chb, cw, nph):
    hw = nheads * cw
    nch = ET // chb

    def body(src_h, dst_h, xw_h, ex_h, inv_h, z_h, p_h,
             sidx0, didx0, xwbuf0, invb0, exb0,
             sidx1, didx1, xwbuf1, invb1, exb1,
             outb, out_sh, sem0, sem1):
        c = lax.axis_index("c")
        s = lax.axis_index("s")
        wid = c * NS + s
        bufs = ((sidx0, didx0, xwbuf0, invb0, exb0),
                (sidx1, didx1, xwbuf1, invb1, exb1))
        sems = (sem0, sem1)

        def stage(k, ch, par):
            sidx, didx, xwbuf, invb, exb = bufs[par]
            base = wid * ET + ch * chb
            pltpu.sync_copy(src_h.at[pl.ds(base, chb)], sidx)
            pltpu.sync_copy(dst_h.at[pl.ds(base, chb)], didx)
            kn = k * N

            def addk(j, carry):
                sl = pl.ds(j * 16, 16)
                sidx[sl] = sidx[sl] + kn
                return carry

            lax.fori_loop(0, chb // 16, addk, 0)
            pltpu.async_copy(xw_h.at[sidx], xwbuf, sems[par])
            pltpu.async_copy(inv_h.at[didx], invb, sems[par])
            pltpu.async_copy(ex_h.at[pl.ds(base, chb)], exb, sems[par])

        def drain(par):
            sidx, didx, xwbuf, invb, exb = bufs[par]
            pltpu.make_async_copy(xw_h.at[sidx], xwbuf, sems[par]).wait()
            pltpu.make_async_copy(inv_h.at[didx], invb, sems[par]).wait()
            pltpu.make_async_copy(ex_h.at[pl.ds(0, chb)], exb, sems[par]).wait()

        def compute(par):
            sidx, didx, xwbuf, invb, exb = bufs[par]

            def edge(e, carry2):
                w = exb[e, :] * invb[e, :]
                for ci in range(cw // 16):
                    acc = w[0] * xwbuf[e, pl.ds(ci * 16, 16)]
                    for hh in range(1, nheads):
                        acc = acc + w[hh] * xwbuf[e, pl.ds(hh * cw + ci * 16, 16)]
                    outb[e, pl.ds(ci * 16, 16)] = acc
                return carry2

            lax.fori_loop(0, chb, edge, 0)
            pltpu.sync_copy(outb, out_sh.at[didx], add=True)

        def phase(k, carry):
            @pl.when(s < N // ROWS)
            def _():
                sl = pl.ds(s * ROWS, ROWS)
                pltpu.sync_copy(z_h.at[sl], out_sh.at[sl])

            plsc.subcore_barrier()
            stage(k, 0, 0)

            def pair(ch2, carry2):
                for par in (0, 1):
                    ch = 2 * ch2 + par

                    @pl.when(ch + 1 < nch)
                    def _():
                        stage(k, ch + 1, 1 - par)

                    drain(par)
                    compute(par)
                return carry2

            lax.fori_loop(0, nch // 2, pair, 0)
            plsc.subcore_barrier()

            @pl.when(s < N // ROWS)
            def _():
                off = (2 * k + c) * N + s * ROWS
                pltpu.sync_copy(out_sh.at[pl.ds(s * ROWS, ROWS)],
                                p_h.at[pl.ds(off, ROWS)])

            plsc.subcore_barrier()
            return carry

        lax.fori_loop(0, nph, phase, 0)

    dbuf = [
        pltpu.VMEM((chb,), jnp.int32),
        pltpu.VMEM((chb,), jnp.int32),
        pltpu.VMEM((chb, hw), jnp.float32),
        pltpu.VMEM((chb, HP), jnp.float32),
        pltpu.VMEM((chb, HP), jnp.float32),
    ]
    return pl.kernel(
        body,
        mesh=_mesh(),
        out_type=jax.ShapeDtypeStruct((nph * 2 * N, cw), jnp.float32),
        scratch_types=dbuf + dbuf + [
            pltpu.VMEM((chb, cw), jnp.float32),
            pltpu.VMEM_SHARED((N, cw), jnp.float32),
            pltpu.SemaphoreType.DMA,
            pltpu.SemaphoreType.DMA,
        ],
        compiler_params=pltpu.CompilerParams(use_tc_tiling_on_sc=False, needs_layout_passes=False),
    )


def _head_matrix(a, nheads):
    m = jnp.zeros((nheads * C, HP), jnp.float32)
    idx = jnp.arange(nheads * C)
    return m.at[idx, idx // C].set(a.reshape(-1))


def kernel(x, edge_index, edge_attr, params):
    ei = edge_index.astype(jnp.int32)
    loop = jnp.arange(N, dtype=jnp.int32)
    padn = EPAD - E - N
    src = jnp.concatenate([ei[0], loop, jnp.zeros((padn,), jnp.int32)])
    dst = jnp.concatenate([ei[1], loop, jnp.zeros((padn,), jnp.int32)])
    dst8 = (dst[:, None] * 8 + jnp.arange(8, dtype=jnp.int32)).reshape(-1)

    wrs, masks = [], []
    for i, h in enumerate(HEADS):
        we = params['We%d' % i]
        aep = params['ae%d' % i]
        wr = (we.reshape(EDIM, h, C) * aep[None]).sum(-1)
        wrs.append(jnp.pad(wr, ((0, 0), (0, HP - h))))
        masks.append(jnp.where(jnp.arange(HP) < h, 0.0, NEGBIG).astype(jnp.float32))
    wr_cat = jnp.concatenate(wrs, axis=1)
    mask_cat = jnp.concatenate(masks)

    ae_real, ae_sum = _tc_ae(edge_attr, wr_cat, mask_cat)
    ae_self = ae_sum[0] / E
    ae_full = jnp.concatenate([
        ae_real,
        jnp.broadcast_to(ae_self, (N, 3 * HP)),
        jnp.full((padn, 3 * HP), NEGBIG, jnp.float32),
    ], axis=0)

    z8 = jnp.zeros((N * 8,), jnp.float32)
    z32 = jnp.zeros((N, 32), jnp.float32)

    h = None
    xw = a_t = d_t = None
    for i, nh in enumerate(HEADS):
        am = _head_matrix(params['as%d' % i], nh)
        dm = _head_matrix(params['ad%d' % i], nh)
        if i == 0:
            h, xwlo, xwhi, a_t, d_t = _tc_init(x, params['in_W'], params['in_b'],
                                               params['W0'], am, dm)
        else:
            xwlo, xwhi, a_t, d_t = _tc_mm(h, params['W%d' % i], am, dm)

        ae_l = lax.slice_in_dim(ae_full, i * HP, (i + 1) * HP, axis=1)
        ex, dparts = _make_passA()(src, dst, dst8, a_t, d_t, ae_l, z8)
        inv8 = _tc_densum(dparts).reshape(N, 8)
        inv = jnp.concatenate([inv8, jnp.zeros((N, 8), jnp.float32)], axis=1)
        nph, cw = 4, 32
        if nh == 8:
            tabs = [jnp.concatenate(
                        [xwlo[:, 64 * hh + 32 * j:64 * hh + 32 * j + 32]
                         for hh in range(8)], axis=1) for j in range(2)]
            tabs += [jnp.concatenate(
                        [xwhi[:, 64 * hh + 32 * j:64 * hh + 32 * j + 32]
                         for hh in range(8)], axis=1) for j in range(2)]
        else:
            tabs = [t[:, 32 * j:32 * j + 32] for t in (xwlo, xwhi)
                    for j in range(2)]
        xw_cat = jnp.concatenate(tabs, axis=0)
        zz = z32
        p_cat = _make_passB(nh, CHB, cw, nph)(src, dst, xw_cat, ex, inv, zz)
        p0 = jnp.concatenate(
            [p_cat[2 * k * N:(2 * k + 1) * N] for k in range(nph)], axis=1)
        p1 = jnp.concatenate(
            [p_cat[(2 * k + 1) * N:(2 * k + 2) * N] for k in range(nph)], axis=1)
        h = _tc_combine(p0, p1, h, params['b%d' % i],
                        params['g%d' % i], params['beta%d' % i], nh)
    return h


# speedup vs baseline: 1.0502x; 1.0053x over previous
import functools

import jax
import jax.numpy as jnp
from jax import lax
from jax.experimental import pallas as pl
from jax.experimental.pallas import tpu as pltpu
from jax.experimental.pallas import tpu_sc as plsc

N = 10000
E = 320000
D = 128
EDIM = 16
C = 128
HEADS = [8, 8, 1]
NEG_SLOPE = 0.2
EPS = 1e-5

NC = 2
NS = 16
NW = NC * NS
ET = 10368
EPAD = NW * ET
NEGBIG = -1e9
HP = 16
CHA = 96
CHB = 96
ROWS = 1000


@functools.lru_cache(maxsize=None)
def _mesh():
    return plsc.VectorSubcoreMesh(
        core_axis_name="c", subcore_axis_name="s",
        num_cores=NC, num_subcores=NS)


_RB = 1000


def _split_heads(xw, nheads):
    lo = jnp.concatenate([xw[:, h * C:h * C + 64] for h in range(nheads)], axis=1)
    hi = jnp.concatenate([xw[:, h * C + 64:(h + 1) * C] for h in range(nheads)], axis=1)
    return lo, hi


def _mm_body(nheads, h_ref, w_ref, am_ref, dm_ref, lo_ref, hi_ref, a_ref, d_ref):
    xw = jnp.dot(h_ref[...], w_ref[...], preferred_element_type=jnp.float32)
    lo_ref[...], hi_ref[...] = _split_heads(xw, nheads)
    a_ref[...] = jnp.dot(xw, am_ref[...], preferred_element_type=jnp.float32)
    d_ref[...] = jnp.dot(xw, dm_ref[...], preferred_element_type=jnp.float32)


def _tc_mm(h, w, am, dm):
    hc = w.shape[1]
    return pl.pallas_call(
        functools.partial(_mm_body, hc // C),
        grid=(N // _RB,),
        in_specs=[
            pl.BlockSpec((_RB, D), lambda i: (i, 0)),
            pl.BlockSpec((D, hc), lambda i: (0, 0)),
            pl.BlockSpec((hc, HP), lambda i: (0, 0)),
            pl.BlockSpec((hc, HP), lambda i: (0, 0)),
        ],
        out_specs=[
            pl.BlockSpec((_RB, hc // 2), lambda i: (i, 0)),
            pl.BlockSpec((_RB, hc // 2), lambda i: (i, 0)),
            pl.BlockSpec((_RB, HP), lambda i: (i, 0)),
            pl.BlockSpec((_RB, HP), lambda i: (i, 0)),
        ],
        out_shape=[
            jax.ShapeDtypeStruct((N, hc // 2), jnp.float32),
            jax.ShapeDtypeStruct((N, hc // 2), jnp.float32),
            jax.ShapeDtypeStruct((N, HP), jnp.float32),
            jax.ShapeDtypeStruct((N, HP), jnp.float32),
        ],
    )(h, w, am, dm)


def _init_body(nheads, x_ref, iw_ref, ib_ref, w_ref, am_ref, dm_ref,
               h_ref, lo_ref, hi_ref, a_ref, d_ref):
    h = jnp.dot(x_ref[...], iw_ref[...], preferred_element_type=jnp.float32)
    h = h + ib_ref[...]
    h_ref[...] = h
    xw = jnp.dot(h, w_ref[...], preferred_element_type=jnp.float32)
    lo_ref[...], hi_ref[...] = _split_heads(xw, nheads)
    a_ref[...] = jnp.dot(xw, am_ref[...], preferred_element_type=jnp.float32)
    d_ref[...] = jnp.dot(xw, dm_ref[...], preferred_element_type=jnp.float32)


def _tc_init(x, iw, ib, w, am, dm):
    hc = w.shape[1]
    return pl.pallas_call(
        functools.partial(_init_body, hc // C),
        grid=(N // _RB,),
        in_specs=[
            pl.BlockSpec((_RB, D), lambda i: (i, 0)),
            pl.BlockSpec((D, C), lambda i: (0, 0)),
            pl.BlockSpec((1, C), lambda i: (0, 0)),
            pl.BlockSpec((C, hc), lambda i: (0, 0)),
            pl.BlockSpec((hc, HP), lambda i: (0, 0)),
            pl.BlockSpec((hc, HP), lambda i: (0, 0)),
        ],
        out_specs=[
            pl.BlockSpec((_RB, C), lambda i: (i, 0)),
            pl.BlockSpec((_RB, hc // 2), lambda i: (i, 0)),
            pl.BlockSpec((_RB, hc // 2), lambda i: (i, 0)),
            pl.BlockSpec((_RB, HP), lambda i: (i, 0)),
            pl.BlockSpec((_RB, HP), lambda i: (i, 0)),
        ],
        out_shape=[
            jax.ShapeDtypeStruct((N, C), jnp.float32),
            jax.ShapeDtypeStruct((N, hc // 2), jnp.float32),
            jax.ShapeDtypeStruct((N, hc // 2), jnp.float32),
            jax.ShapeDtypeStruct((N, HP), jnp.float32),
            jax.ShapeDtypeStruct((N, HP), jnp.float32),
        ],
    )(x, iw, ib.reshape(1, C), w, am, dm)


_AEB = 2000


def _ae_body(ea_ref, wr_ref, mask_ref, ae_ref, sum_ref):
    o = jnp.dot(ea_ref[...], wr_ref[...], preferred_element_type=jnp.float32)
    o = o + mask_ref[...]
    ae_ref[...] = o

    @pl.when(pl.program_id(0) == 0)
    def _():
        sum_ref[...] = jnp.zeros_like(sum_ref)

    sum_ref[...] += jnp.sum(o, axis=0, keepdims=True)


def _tc_ae(ea, wr_cat, mask_cat):
    w3 = wr_cat.shape[1]
    return pl.pallas_call(
        _ae_body,
        grid=(E // _AEB,),
        in_specs=[
            pl.BlockSpec((_AEB, EDIM), lambda i: (i, 0)),
            pl.BlockSpec((EDIM, w3), lambda i: (0, 0)),
            pl.BlockSpec((1, w3), lambda i: (0, 0)),
        ],
        out_specs=[
            pl.BlockSpec((_AEB, w3), lambda i: (i, 0)),
            pl.BlockSpec((1, w3), lambda i: (0, 0)),
        ],
        out_shape=[
            jax.ShapeDtypeStruct((E, w3), jnp.float32),
            jax.ShapeDtypeStruct((1, w3), jnp.float32),
        ],
    )(ea, wr_cat, mask_cat.reshape(1, w3))


_DB = 16000


def _densum_body(dp_ref, inv_ref):
    s = jnp.sum(dp_ref[...], axis=0)
    inv_ref[...] = 1.0 / (s + 1e-16)


def _tc_densum(dparts):
    return pl.pallas_call(
        _densum_body,
        grid=(1,),
        in_specs=[pl.BlockSpec((NW, N * 8), lambda i: (0, 0))],
        out_specs=pl.BlockSpec((N * 8,), lambda i: (0,)),
        out_shape=jax.ShapeDtypeStruct((N * 8,), jnp.float32),
    )(dparts)


def _combine_body(inv_h, p0_ref, p1_ref, res_ref, b_ref, g_ref, bb_ref,
                  out_ref):
    o = (p0_ref[...] + p1_ref[...]) * inv_h + b_ref[...]
    mu = jnp.mean(o, axis=0, keepdims=True)
    var = jnp.mean(o * o, axis=0, keepdims=True) - mu * mu
    o = (o - mu) * lax.rsqrt(var + EPS) * g_ref[...] + bb_ref[...]
    o = jnp.where(o > 0, o, jnp.exp(o) - 1.0)
    out_ref[...] = o + res_ref[...]


def _tc_combine(p0, p1, res, b, g, bb, nheads):
    body = functools.partial(_combine_body, 1.0 / nheads)
    return pl.pallas_call(
        body,
        grid=(1,),
        in_specs=[
            pl.BlockSpec((N, C), lambda i: (0, 0)),
            pl.BlockSpec((N, C), lambda i: (0, 0)),
            pl.BlockSpec((N, C), lambda i: (0, 0)),
            pl.BlockSpec((1, C), lambda i: (0, 0)),
            pl.BlockSpec((1, C), lambda i: (0, 0)),
            pl.BlockSpec((1, C), lambda i: (0, 0)),
        ],
        out_specs=pl.BlockSpec((N, C), lambda i: (0, 0)),
        out_shape=jax.ShapeDtypeStruct((N, C), jnp.float32),
    )(p0, p1, res, b.reshape(1, C), g.reshape(1, C), bb.reshape(1, C))


def _passA_body(src_h, dst_h, d8_h, as_h, ad_h, ae_h, z8_h,
                ex_h, dp_h,
                sidx0, didx0, d8b0, bs0, bd0, be0,
                sidx1, didx1, d8b1, bs1, bd1, be1,
                den8, sem0, sem1):
    c = lax.axis_index("c")
    s = lax.axis_index("s")
    wid = c * NS + s
    bufs = ((sidx0, didx0, d8b0, bs0, bd0, be0),
            (sidx1, didx1, d8b1, bs1, bd1, be1))
    sems = (sem0, sem1)
    nch = ET // CHA
    pltpu.sync_copy(z8_h, den8)
    lanes = lax.iota(jnp.int32, 16)
    msk8 = lanes < 8

    def stage(ch, par):
        sidx, didx, d8buf, buf_s, buf_d, buf_e = bufs[par]
        base = wid * ET + ch * CHA
        pltpu.sync_copy(src_h.at[pl.ds(base, CHA)], sidx)
        pltpu.sync_copy(dst_h.at[pl.ds(base, CHA)], didx)
        pltpu.async_copy(d8_h.at[pl.ds(base * 8, CHA * 8)],
                         d8buf.at[pl.ds(0, CHA * 8)], sems[par])
        pltpu.async_copy(as_h.at[sidx], buf_s, sems[par])
        pltpu.async_copy(ad_h.at[didx], buf_d, sems[par])
        pltpu.async_copy(ae_h.at[pl.ds(base, CHA)], buf_e, sems[par])

    def drain(par):
        sidx, didx, d8buf, buf_s, buf_d, buf_e = bufs[par]
        pltpu.make_async_copy(d8_h.at[pl.ds(0, CHA * 8)],
                              d8buf.at[pl.ds(0, CHA * 8)], sems[par]).wait()
        pltpu.make_async_copy(as_h.at[sidx], buf_s, sems[par]).wait()
        pltpu.make_async_copy(ad_h.at[didx], buf_d, sems[par]).wait()
        pltpu.make_async_copy(ae_h.at[pl.ds(0, CHA)], buf_e, sems[par]).wait()

    def compute(ch, par):
        sidx, didx, d8buf, buf_s, buf_d, buf_e = bufs[par]
        base = wid * ET + ch * CHA

        def edge(e, carry2):
            a = buf_s[e, :] + buf_d[e, :] + buf_e[e, :]
            a = jnp.where(a >= 0, a, jnp.float32(NEG_SLOPE) * a)
            ex = jnp.exp(a)
            buf_e[e, :] = ex
            idxd = d8buf[pl.ds(e * 8, 16)]
            plsc.addupdate_scatter(den8, [idxd], ex, mask=msk8)
            return carry2

        lax.fori_loop(0, CHA, edge, 0, unroll=4)
        pltpu.sync_copy(buf_e, ex_h.at[pl.ds(base, CHA)])

    stage(0, 0)

    def pair(ch2, carry):
        for par in (0, 1):
            ch = 2 * ch2 + par

            @pl.when(ch + 1 < nch)
            def _():
                stage(ch + 1, 1 - par)

            drain(par)
            compute(ch, par)
        return carry

    lax.fori_loop(0, nch // 2, pair, 0)
    pltpu.sync_copy(den8, dp_h.at[wid])


@functools.lru_cache(maxsize=None)
def _make_passA():
    abuf = [
        pltpu.VMEM((CHA,), jnp.int32),
        pltpu.VMEM((CHA,), jnp.int32),
        pltpu.VMEM((CHA * 8 + 16,), jnp.int32),
        pltpu.VMEM((CHA, HP), jnp.float32),
        pltpu.VMEM((CHA, HP), jnp.float32),
        pltpu.VMEM((CHA, HP), jnp.float32),
    ]
    return pl.kernel(
        _passA_body,
        mesh=_mesh(),
        out_type=[
            jax.ShapeDtypeStruct((EPAD, HP), jnp.float32),
            jax.ShapeDtypeStruct((NW, N * 8), jnp.float32),
        ],
        scratch_types=abuf + abuf + [
            pltpu.VMEM((N * 8,), jnp.float32),
            pltpu.SemaphoreType.DMA,
            pltpu.SemaphoreType.DMA,
        ],
        compiler_params=pltpu.CompilerParams(use_tc_tiling_on_sc=False, needs_layout_passes=False),
    )


@functools.lru_cache(maxsize=None)
def _make_passB(nheads, chb, cw, nph):
    hw = nheads * cw
    nch = ET // chb

    def body(src_h, dst_h, xw_h, ex_h, inv_h, z_h, p_h,
             sidx0, didx0, xwbuf0, invb0, exb0,
             sidx1, didx1, xwbuf1, invb1, exb1,
             outb, out_sh, sem0, sem1):
        c = lax.axis_index("c")
        s = lax.axis_index("s")
        wid = c * NS + s
        bufs = ((sidx0, didx0, xwbuf0, invb0, exb0),
                (sidx1, didx1, xwbuf1, invb1, exb1))
        sems = (sem0, sem1)

        def stage(k, ch, par):
            sidx, didx, xwbuf, invb, exb = bufs[par]
            base = wid * ET + ch * chb
            pltpu.sync_copy(src_h.at[pl.ds(base, chb)], sidx)
            pltpu.sync_copy(dst_h.at[pl.ds(base, chb)], didx)
            kn = k * N

            def addk(j, carry):
                sl = pl.ds(j * 16, 16)
                sidx[sl] = sidx[sl] + kn
                return carry

            lax.fori_loop(0, chb // 16, addk, 0)
            pltpu.async_copy(xw_h.at[sidx], xwbuf, sems[par])
            pltpu.async_copy(inv_h.at[didx], invb, sems[par])
            pltpu.async_copy(ex_h.at[pl.ds(base, chb)], exb, sems[par])

        def drain(par):
            sidx, didx, xwbuf, invb, exb = bufs[par]
            pltpu.make_async_copy(xw_h.at[sidx], xwbuf, sems[par]).wait()
            pltpu.make_async_copy(inv_h.at[didx], invb, sems[par]).wait()
            pltpu.make_async_copy(ex_h.at[pl.ds(0, chb)], exb, sems[par]).wait()

        def compute(par):
            sidx, didx, xwbuf, invb, exb = bufs[par]

            def edge(e, carry2):
                w = exb[e, :] * invb[e, :]
                for ci in range(cw // 16):
                    acc = w[0] * xwbuf[e, pl.ds(ci * 16, 16)]
                    for hh in range(1, nheads):
                        acc = acc + w[hh] * xwbuf[e, pl.ds(hh * cw + ci * 16, 16)]
                    outb[e, pl.ds(ci * 16, 16)] = acc
                return carry2

            lax.fori_loop(0, chb, edge, 0, unroll=4)
            pltpu.sync_copy(outb, out_sh.at[didx], add=True)

        def phase(k, carry):
            @pl.when(s < N // ROWS)
            def _():
                sl = pl.ds(s * ROWS, ROWS)
                pltpu.sync_copy(z_h.at[sl], out_sh.at[sl])

            plsc.subcore_barrier()
            stage(k, 0, 0)

            def pair(ch2, carry2):
                for par in (0, 1):
                    ch = 2 * ch2 + par

                    @pl.when(ch + 1 < nch)
                    def _():
                        stage(k, ch + 1, 1 - par)

                    drain(par)
                    compute(par)
                return carry2

            lax.fori_loop(0, nch // 2, pair, 0)
            plsc.subcore_barrier()

            @pl.when(s < N // ROWS)
            def _():
                off = (2 * k + c) * N + s * ROWS
                pltpu.sync_copy(out_sh.at[pl.ds(s * ROWS, ROWS)],
                                p_h.at[pl.ds(off, ROWS)])

            plsc.subcore_barrier()
            return carry

        lax.fori_loop(0, nph, phase, 0)

    dbuf = [
        pltpu.VMEM((chb,), jnp.int32),
        pltpu.VMEM((chb,), jnp.int32),
        pltpu.VMEM((chb, hw), jnp.float32),
        pltpu.VMEM((chb, HP), jnp.float32),
        pltpu.VMEM((chb, HP), jnp.float32),
    ]
    return pl.kernel(
        body,
        mesh=_mesh(),
        out_type=jax.ShapeDtypeStruct((nph * 2 * N, cw), jnp.float32),
        scratch_types=dbuf + dbuf + [
            pltpu.VMEM((chb, cw), jnp.float32),
            pltpu.VMEM_SHARED((N, cw), jnp.float32),
            pltpu.SemaphoreType.DMA,
            pltpu.SemaphoreType.DMA,
        ],
        compiler_params=pltpu.CompilerParams(use_tc_tiling_on_sc=False, needs_layout_passes=False),
    )


def _head_matrix(a, nheads):
    m = jnp.zeros((nheads * C, HP), jnp.float32)
    idx = jnp.arange(nheads * C)
    return m.at[idx, idx // C].set(a.reshape(-1))


def kernel(x, edge_index, edge_attr, params):
    ei = edge_index.astype(jnp.int32)
    loop = jnp.arange(N, dtype=jnp.int32)
    padn = EPAD - E - N
    src = jnp.concatenate([ei[0], loop, jnp.zeros((padn,), jnp.int32)])
    dst = jnp.concatenate([ei[1], loop, jnp.zeros((padn,), jnp.int32)])
    dst8 = (dst[:, None] * 8 + jnp.arange(8, dtype=jnp.int32)).reshape(-1)

    wrs, masks = [], []
    for i, h in enumerate(HEADS):
        we = params['We%d' % i]
        aep = params['ae%d' % i]
        wr = (we.reshape(EDIM, h, C) * aep[None]).sum(-1)
        wrs.append(jnp.pad(wr, ((0, 0), (0, HP - h))))
        masks.append(jnp.where(jnp.arange(HP) < h, 0.0, NEGBIG).astype(jnp.float32))
    wr_cat = jnp.concatenate(wrs, axis=1)
    mask_cat = jnp.concatenate(masks)

    ae_real, ae_sum = _tc_ae(edge_attr, wr_cat, mask_cat)
    ae_self = ae_sum[0] / E
    ae_full = jnp.concatenate([
        ae_real,
        jnp.broadcast_to(ae_self, (N, 3 * HP)),
        jnp.full((padn, 3 * HP), NEGBIG, jnp.float32),
    ], axis=0)

    z8 = jnp.zeros((N * 8,), jnp.float32)
    z32 = jnp.zeros((N, 32), jnp.float32)

    h = None
    xw = a_t = d_t = None
    for i, nh in enumerate(HEADS):
        am = _head_matrix(params['as%d' % i], nh)
        dm = _head_matrix(params['ad%d' % i], nh)
        if i == 0:
            h, xwlo, xwhi, a_t, d_t = _tc_init(x, params['in_W'], params['in_b'],
                                               params['W0'], am, dm)
        else:
            xwlo, xwhi, a_t, d_t = _tc_mm(h, params['W%d' % i], am, dm)

        ae_l = lax.slice_in_dim(ae_full, i * HP, (i + 1) * HP, axis=1)
        ex, dparts = _make_passA()(src, dst, dst8, a_t, d_t, ae_l, z8)
        inv8 = _tc_densum(dparts).reshape(N, 8)
        inv = jnp.concatenate([inv8, jnp.zeros((N, 8), jnp.float32)], axis=1)
        nph, cw = 4, 32
        if nh == 8:
            tabs = [jnp.concatenate(
                        [xwlo[:, 64 * hh + 32 * j:64 * hh + 32 * j + 32]
                         for hh in range(8)], axis=1) for j in range(2)]
            tabs += [jnp.concatenate(
                        [xwhi[:, 64 * hh + 32 * j:64 * hh + 32 * j + 32]
                         for hh in range(8)], axis=1) for j in range(2)]
        else:
            tabs = [t[:, 32 * j:32 * j + 32] for t in (xwlo, xwhi)
                    for j in range(2)]
        xw_cat = jnp.concatenate(tabs, axis=0)
        zz = z32
        p_cat = _make_passB(nh, CHB, cw, nph)(src, dst, xw_cat, ex, inv, zz)
        p0 = jnp.concatenate(
            [p_cat[2 * k * N:(2 * k + 1) * N] for k in range(nph)], axis=1)
        p1 = jnp.concatenate(
            [p_cat[(2 * k + 1) * N:(2 * k + 2) * N] for k in range(nph)], axis=1)
        h = _tc_combine(p0, p1, h, params['b%d' % i],
                        params['g%d' % i], params['beta%d' % i], nh)
    return h
